# baseline (speedup 1.0000x reference)
import math

import jax
import jax.numpy as jnp
from jax.experimental import pallas as pl
from jax.experimental.pallas import tpu as pltpu

_B, _L, _D, _P, _H = 2, 2048, 256, 32, 4
_C = 1024
_NC = _L // _C
_TP = 2 * _P
_DH = _D // 2
_F = _D + _P

_PREC = jax.lax.Precision.DEFAULT

_SIN_C = (0.9999999561919739, -0.16666631903685214, 0.008332890518018223,
          -0.0001982075315889011, 2.712795502476465e-06, -2.087246249914055e-08)
_COS_C = (0.9999999922855538, -0.4999999176805242, 0.04166652429874245,
          -0.00138879700727022, 2.4773416495296434e-05, -2.7113293532897174e-07,
          1.736882787082075e-09)


def _sincos_bounded(x):
    u = x * x
    s = _SIN_C[5]
    for a in _SIN_C[4::-1]:
        s = s * u + a
    s = s * x
    c = _COS_C[6]
    for a in _COS_C[5::-1]:
        c = c * u + a
    return c, s


def _dot(a, b):
    return jax.lax.dot_general(a, b, (((1,), (0,)), ((), ())),
                               preferred_element_type=jnp.float32,
                               precision=_PREC)


def _dot_tn(a, b):
    return jax.lax.dot_general(a, b, (((0,), (0,)), ((), ())),
                               preferred_element_type=jnp.float32,
                               precision=_PREC)


def _dot_nt(a, b):
    return jax.lax.dot_general(a, b, (((1,), (1,)), ((), ())),
                               preferred_element_type=jnp.float32,
                               precision=_PREC)


def _gelu(t):
    return 0.5 * t * (1.0 + jax.lax.erf(t * (1.0 / math.sqrt(2.0))))


def _phasor_kernel(x_ref, pp_ref,
                   wc_ref, bc_ref,
                   hw1_ref, hb1_ref, hw2_ref, hb2_ref, hb2s_ref,
                   wcpe_ref, bcpe_ref, wval_ref, bval_ref, wctx_ref, bctx_ref,
                   kw1_ref, kb1_ref, kw2_ref, kb2_ref,
                   sw1_ref, sb1_ref, sw2_ref, sb2_ref,
                   lng_ref, lnb_ref, wout_ref, bout_ref,
                   out_ref,
                   s1_ref, s2_ref, ctxc_ref, gatec_ref, tril_ref):
    c = pl.program_id(1)

    @pl.when(c == 0)
    def _():
        s1_ref[...] = jnp.zeros_like(s1_ref)
        s2_ref[...] = jnp.zeros_like(s2_ref)
        ctxc_ref[...] = jnp.zeros_like(ctxc_ref)
        gatec_ref[...] = jnp.zeros_like(gatec_ref)
        ri = jax.lax.broadcasted_iota(jnp.int32, (_C, _C), 0)
        ci = jax.lax.broadcasted_iota(jnp.int32, (_C, _C), 1)
        tril_ref[...] = jnp.where(ri >= ci, 1.0, 0.0)

    xb = x_ref[0]
    pp = pp_ref[...]
    tril = tril_ref[...]
    k = jnp.round(pp * (1.0 / (2.0 * math.pi)))
    ppr = pp - k * (2.0 * math.pi)
    pc, ps = _sincos_bounded(ppr)
    posk = jnp.concatenate([pc, ps], axis=1)
    poskt = jnp.swapaxes(posk, 0, 1)

    content = _dot_nt(xb, wc_ref[...]) + bc_ref[...]
    oi = jnp.concatenate([xb, pp], axis=1)

    zs = []
    wws = []
    for h in range(_H):
        h1h = _gelu(_dot_nt(oi, hw1_ref[h]) + hb1_ref[h:h + 1])
        zs.append(_dot_nt(h1h, hw2_ref[h, :_P]) + hb2_ref[h:h + 1, :_P])
        wws.append(jnp.sum(h1h * hw2_ref[h, _P:_P + 1], axis=1, keepdims=True)
                   + hb2s_ref[h, _P])
    z = jnp.tanh(jnp.concatenate(zs, axis=1)) * math.pi
    ww4 = jnp.concatenate(wws, axis=1)

    ww = jax.nn.sigmoid(ww4) * 2.0 + 0.1
    r = 1.0 / (ww * ww + 1e-6)
    w1 = jnp.exp(-0.5 * r)
    w2 = w1 * w1
    w2 = w2 * w2
    c1 = math.cos(0.1)
    c2 = math.cos(0.2)
    scale = 1.0 / (_H * math.sqrt(_P))
    g = (scale * (1.0 + 2.0 * c1 * w1 + 2.0 * c2 * w2)
         / (1.0 + 2.0 * w1 + 2.0 * w2 + 1e-6))

    e4 = (jax.lax.broadcasted_iota(jnp.int32, (_H, _H * _P), 1) // _P
          == jax.lax.broadcasted_iota(jnp.int32, (_H, _H * _P), 0))
    e4 = jnp.where(e4, 1.0, 0.0)
    m32 = (jax.lax.broadcasted_iota(jnp.int32, (_H * _P, _P), 0) % _P
           == jax.lax.broadcasted_iota(jnp.int32, (_H * _P, _P), 1))
    m32 = jnp.where(m32, 1.0, 0.0)
    g128 = _dot(g, e4)

    cz, sz = _sincos_bounded(z)
    qa = _dot(g128 * cz, m32)
    qb = _dot(g128 * sz, m32)

    qeff = jnp.concatenate([pc * qa - ps * qb, ps * qa + pc * qb], axis=1)

    scores1 = _dot(qeff, poskt) * tril
    mh = _dot(scores1, content) + _dot(qeff, s1_ref[...])
    s1_ref[...] = s1_ref[...] + _dot(poskt, content)

    key_ph = jnp.tanh(_dot_nt(xb, wcpe_ref[...]) + bcpe_ref[...]) * math.pi
    values = _dot_nt(xb, wval_ref[...]) + bval_ref[...]
    sgh = _gelu(_dot_nt(xb, sw1_ref[...]) + sb1_ref[...])
    sgate = jax.nn.sigmoid(jnp.sum(sgh * sw2_ref[...], axis=1, keepdims=True)
                           + sb2_ref[...])
    ctx = _dot_nt(xb, wctx_ref[...]) + bctx_ref[...]

    ctx_cum = _dot(tril, ctx) + ctxc_ref[...]
    pos = (jax.lax.broadcasted_iota(jnp.int32, (_C, 1), 0) + (c * _C + 1)).astype(jnp.float32)
    ctx_avg = ctx_cum * (1.0 / pos)
    ctxc_ref[...] = ctx_cum[_C - 1:_C, :]

    si = jnp.concatenate([xb, ctx_avg], axis=1)
    sp = jnp.tanh(_dot_nt(_gelu(_dot_nt(si, kw1_ref[...]) + kb1_ref[...]),
                          kw2_ref[...]) + kb2_ref[...]) * math.pi

    gcum = _dot(tril, sgate) + gatec_ref[...]
    gatec_ref[...] = gcum[_C - 1:_C, :]

    trig_in = jnp.concatenate([key_ph, sp], axis=1)
    ct, st = _sincos_bounded(trig_in)
    q2 = jnp.concatenate([ct[:, :_P], st[:, :_P]], axis=1)
    k2 = jnp.concatenate([ct[:, _P:], st[:, _P:]], axis=1)
    gv = values * sgate

    scores2 = _dot_nt(q2, k2) * tril
    kv_ret = _dot(scores2, gv) + _dot(q2, s2_ref[...])
    kv_ret = kv_ret * (jax.lax.rsqrt(jnp.maximum(gcum, 1.0)) * (1.0 / math.sqrt(_P)))
    s2_ref[...] = s2_ref[...] + _dot_tn(k2, gv)

    combined = mh + kv_ret
    mu = jnp.mean(combined, axis=1, keepdims=True)
    dlt = combined - mu
    var = jnp.mean(dlt * dlt, axis=1, keepdims=True)
    ln = dlt * jax.lax.rsqrt(var + 1e-5) * lng_ref[...] + lnb_ref[...]
    out_ref[0] = xb + _dot_nt(ln, wout_ref[...]) + bout_ref[...]


def _full(shape):
    return pl.BlockSpec(shape, lambda b, c: (0,) * len(shape))


def kernel(x, pos_phases, w_content, b_content, hop_w1, hop_b1, hop_w2, hop_b2,
           w_cpe, b_cpe, w_val, b_val, w_ctx, b_ctx, kv_w1, kv_b1, kv_w2, kv_b2,
           sg_w1, sg_b1, sg_w2, sg_b2, ln_g, ln_b, w_out, b_out):
    args = (
        x, pos_phases,
        w_content, b_content[None],
        hop_w1, hop_b1,
        hop_w2, hop_b2, hop_b2,
        w_cpe, b_cpe[None],
        w_val, b_val[None],
        w_ctx, b_ctx[None],
        kv_w1, kv_b1[None],
        kv_w2, kv_b2[None],
        sg_w1, sg_b1[None],
        sg_w2, sg_b2[None],
        ln_g[None], ln_b[None],
        w_out, b_out[None],
    )

    in_specs = [
        pl.BlockSpec((1, _C, _D), lambda b, c: (b, c, 0)),
        pl.BlockSpec((_C, _P), lambda b, c: (c, 0)),
        _full((_D, _D)), _full((1, _D)),
        _full((_H, _DH, _F)), _full((_H, _DH)),
        _full((_H, _P + 1, _DH)), _full((_H, _P + 1)),
        pl.BlockSpec(memory_space=pltpu.SMEM),
        _full((_P, _D)), _full((1, _P)),
        _full((_D, _D)), _full((1, _D)),
        _full((_D, _D)), _full((1, _D)),
        _full((_D, 2 * _D)), _full((1, _D)),
        _full((_P, _D)), _full((1, _P)),
        _full((_DH, _D)), _full((1, _DH)),
        _full((1, _DH)), _full((1, 1)),
        _full((1, _D)), _full((1, _D)),
        _full((_D, _D)), _full((1, _D)),
    ]

    out = pl.pallas_call(
        _phasor_kernel,
        grid=(_B, _NC),
        in_specs=in_specs,
        out_specs=pl.BlockSpec((1, _C, _D), lambda b, c: (b, c, 0)),
        out_shape=jax.ShapeDtypeStruct((_B, _L, _D), jnp.float32),
        scratch_shapes=[
            pltpu.VMEM((_TP, _D), jnp.float32),
            pltpu.VMEM((_TP, _D), jnp.float32),
            pltpu.VMEM((1, _D), jnp.float32),
            pltpu.VMEM((1, 1), jnp.float32),
            pltpu.VMEM((_C, _C), jnp.float32),
        ],
        compiler_params=pltpu.CompilerParams(
            dimension_semantics=("parallel", "arbitrary"),
        ),
    )(*args)
    return out


# speedup vs baseline: 1.1753x; 1.1753x over previous
import math

import jax
import jax.numpy as jnp
from jax.experimental import pallas as pl
from jax.experimental.pallas import tpu as pltpu

_B, _L, _D, _P, _H = 2, 2048, 256, 32, 4
_C = 512
_NC = _L // _C
_TP = 2 * _P
_DH = _D // 2
_F = _D + _P

_PREC = jax.lax.Precision.DEFAULT

_SIN_C = (0.9999999561919739, -0.16666631903685214, 0.008332890518018223,
          -0.0001982075315889011, 2.712795502476465e-06, -2.087246249914055e-08)
_COS_C = (0.9999999922855538, -0.4999999176805242, 0.04166652429874245,
          -0.00138879700727022, 2.4773416495296434e-05, -2.7113293532897174e-07,
          1.736882787082075e-09)


def _sincos_bounded(x):
    u = x * x
    s = _SIN_C[5]
    for a in _SIN_C[4::-1]:
        s = s * u + a
    s = s * x
    c = _COS_C[6]
    for a in _COS_C[5::-1]:
        c = c * u + a
    return c, s


def _dot(a, b):
    return jax.lax.dot_general(a, b, (((1,), (0,)), ((), ())),
                               preferred_element_type=jnp.float32,
                               precision=_PREC)


def _dot_tn(a, b):
    return jax.lax.dot_general(a, b, (((0,), (0,)), ((), ())),
                               preferred_element_type=jnp.float32,
                               precision=_PREC)


def _dot_nt(a, b):
    return jax.lax.dot_general(a, b, (((1,), (1,)), ((), ())),
                               preferred_element_type=jnp.float32,
                               precision=_PREC)


def _gelu(t):
    return 0.5 * t * (1.0 + jax.lax.erf(t * (1.0 / math.sqrt(2.0))))


def _phasor_kernel(x_ref, pp_ref,
                   wc_ref, bc_ref,
                   hw1_ref, hb1_ref, hw2_ref, hb2_ref, hb2s_ref,
                   wcpe_ref, bcpe_ref, wval_ref, bval_ref, wctx_ref, bctx_ref,
                   kw1_ref, kb1_ref, kw2_ref, kb2_ref,
                   sw1_ref, sb1_ref, sw2_ref, sb2_ref,
                   lng_ref, lnb_ref, wout_ref, bout_ref,
                   out_ref,
                   s1_ref, s2_ref, ctxc_ref, gatec_ref, tril_ref):
    c = pl.program_id(1)

    @pl.when(c == 0)
    def _():
        s1_ref[...] = jnp.zeros_like(s1_ref)
        s2_ref[...] = jnp.zeros_like(s2_ref)
        ctxc_ref[...] = jnp.zeros_like(ctxc_ref)
        gatec_ref[...] = jnp.zeros_like(gatec_ref)
        ri = jax.lax.broadcasted_iota(jnp.int32, (_C, _C), 0)
        ci = jax.lax.broadcasted_iota(jnp.int32, (_C, _C), 1)
        tril_ref[...] = jnp.where(ri >= ci, 1.0, 0.0)

    xb = x_ref[0]
    pp = pp_ref[...]
    tril = tril_ref[...]
    k = jnp.round(pp * (1.0 / (2.0 * math.pi)))
    ppr = pp - k * (2.0 * math.pi)
    pc, ps = _sincos_bounded(ppr)
    posk = jnp.concatenate([pc, ps], axis=1)
    poskt = jnp.swapaxes(posk, 0, 1)

    content = _dot_nt(xb, wc_ref[...]) + bc_ref[...]
    oi = jnp.concatenate([xb, pp], axis=1)

    zs = []
    wws = []
    for h in range(_H):
        h1h = _gelu(_dot_nt(oi, hw1_ref[h]) + hb1_ref[h:h + 1])
        zs.append(_dot_nt(h1h, hw2_ref[h, :_P]) + hb2_ref[h:h + 1, :_P])
        wws.append(jnp.sum(h1h * hw2_ref[h, _P:_P + 1], axis=1, keepdims=True)
                   + hb2s_ref[h, _P])
    z = jnp.tanh(jnp.concatenate(zs, axis=1)) * math.pi
    ww4 = jnp.concatenate(wws, axis=1)

    ww = jax.nn.sigmoid(ww4) * 2.0 + 0.1
    r = 1.0 / (ww * ww + 1e-6)
    w1 = jnp.exp(-0.5 * r)
    w2 = w1 * w1
    w2 = w2 * w2
    c1 = math.cos(0.1)
    c2 = math.cos(0.2)
    scale = 1.0 / (_H * math.sqrt(_P))
    g = (scale * (1.0 + 2.0 * c1 * w1 + 2.0 * c2 * w2)
         / (1.0 + 2.0 * w1 + 2.0 * w2 + 1e-6))

    e4 = (jax.lax.broadcasted_iota(jnp.int32, (_H, _H * _P), 1) // _P
          == jax.lax.broadcasted_iota(jnp.int32, (_H, _H * _P), 0))
    e4 = jnp.where(e4, 1.0, 0.0)
    m32 = (jax.lax.broadcasted_iota(jnp.int32, (_H * _P, _P), 0) % _P
           == jax.lax.broadcasted_iota(jnp.int32, (_H * _P, _P), 1))
    m32 = jnp.where(m32, 1.0, 0.0)
    g128 = _dot(g, e4)

    cz, sz = _sincos_bounded(z)
    qa = _dot(g128 * cz, m32)
    qb = _dot(g128 * sz, m32)

    qeff = jnp.concatenate([pc * qa - ps * qb, ps * qa + pc * qb], axis=1)

    scores1 = _dot(qeff, poskt) * tril
    mh = _dot(scores1, content) + _dot(qeff, s1_ref[...])
    s1_ref[...] = s1_ref[...] + _dot(poskt, content)

    key_ph = jnp.tanh(_dot_nt(xb, wcpe_ref[...]) + bcpe_ref[...]) * math.pi
    values = _dot_nt(xb, wval_ref[...]) + bval_ref[...]
    sgh = _gelu(_dot_nt(xb, sw1_ref[...]) + sb1_ref[...])
    sgate = jax.nn.sigmoid(jnp.sum(sgh * sw2_ref[...], axis=1, keepdims=True)
                           + sb2_ref[...])
    ctx = _dot_nt(xb, wctx_ref[...]) + bctx_ref[...]

    ctx_cum = _dot(tril, ctx) + ctxc_ref[...]
    pos = (jax.lax.broadcasted_iota(jnp.int32, (_C, 1), 0) + (c * _C + 1)).astype(jnp.float32)
    ctx_avg = ctx_cum * (1.0 / pos)
    ctxc_ref[...] = ctx_cum[_C - 1:_C, :]

    si = jnp.concatenate([xb, ctx_avg], axis=1)
    sp = jnp.tanh(_dot_nt(_gelu(_dot_nt(si, kw1_ref[...]) + kb1_ref[...]),
                          kw2_ref[...]) + kb2_ref[...]) * math.pi

    gcum = _dot(tril, sgate) + gatec_ref[...]
    gatec_ref[...] = gcum[_C - 1:_C, :]

    trig_in = jnp.concatenate([key_ph, sp], axis=1)
    ct, st = _sincos_bounded(trig_in)
    q2 = jnp.concatenate([ct[:, :_P], st[:, :_P]], axis=1)
    k2 = jnp.concatenate([ct[:, _P:], st[:, _P:]], axis=1)
    gv = values * sgate

    scores2 = _dot_nt(q2, k2) * tril
    kv_ret = _dot(scores2, gv) + _dot(q2, s2_ref[...])
    kv_ret = kv_ret * (jax.lax.rsqrt(jnp.maximum(gcum, 1.0)) * (1.0 / math.sqrt(_P)))
    s2_ref[...] = s2_ref[...] + _dot_tn(k2, gv)

    combined = mh + kv_ret
    mu = jnp.mean(combined, axis=1, keepdims=True)
    dlt = combined - mu
    var = jnp.mean(dlt * dlt, axis=1, keepdims=True)
    ln = dlt * jax.lax.rsqrt(var + 1e-5) * lng_ref[...] + lnb_ref[...]
    out_ref[0] = xb + _dot_nt(ln, wout_ref[...]) + bout_ref[...]


def _full(shape):
    return pl.BlockSpec(shape, lambda b, c: (0,) * len(shape))


def kernel(x, pos_phases, w_content, b_content, hop_w1, hop_b1, hop_w2, hop_b2,
           w_cpe, b_cpe, w_val, b_val, w_ctx, b_ctx, kv_w1, kv_b1, kv_w2, kv_b2,
           sg_w1, sg_b1, sg_w2, sg_b2, ln_g, ln_b, w_out, b_out):
    args = (
        x, pos_phases[:_L],
        w_content, b_content[None],
        hop_w1, hop_b1,
        hop_w2, hop_b2, hop_b2,
        w_cpe, b_cpe[None],
        w_val, b_val[None],
        w_ctx, b_ctx[None],
        kv_w1, kv_b1[None],
        kv_w2, kv_b2[None],
        sg_w1, sg_b1[None],
        sg_w2, sg_b2[None],
        ln_g[None], ln_b[None],
        w_out, b_out[None],
    )

    in_specs = [
        pl.BlockSpec((1, _C, _D), lambda b, c: (b, c, 0)),
        pl.BlockSpec((_C, _P), lambda b, c: (c, 0)),
        _full((_D, _D)), _full((1, _D)),
        _full((_H, _DH, _F)), _full((_H, _DH)),
        _full((_H, _P + 1, _DH)), _full((_H, _P + 1)),
        pl.BlockSpec(memory_space=pltpu.SMEM),
        _full((_P, _D)), _full((1, _P)),
        _full((_D, _D)), _full((1, _D)),
        _full((_D, _D)), _full((1, _D)),
        _full((_D, 2 * _D)), _full((1, _D)),
        _full((_P, _D)), _full((1, _P)),
        _full((_DH, _D)), _full((1, _DH)),
        _full((1, _DH)), _full((1, 1)),
        _full((1, _D)), _full((1, _D)),
        _full((_D, _D)), _full((1, _D)),
    ]

    out = pl.pallas_call(
        _phasor_kernel,
        grid=(_B, _NC),
        in_specs=in_specs,
        out_specs=pl.BlockSpec((1, _C, _D), lambda b, c: (b, c, 0)),
        out_shape=jax.ShapeDtypeStruct((_B, _L, _D), jnp.float32),
        scratch_shapes=[
            pltpu.VMEM((_TP, _D), jnp.float32),
            pltpu.VMEM((_TP, _D), jnp.float32),
            pltpu.VMEM((1, _D), jnp.float32),
            pltpu.VMEM((1, 1), jnp.float32),
            pltpu.VMEM((_C, _C), jnp.float32),
        ],
        compiler_params=pltpu.CompilerParams(
            dimension_semantics=("parallel", "arbitrary"),
        ),
    )(*args)
    return out
